# Initial kernel scaffold; baseline (speedup 1.0000x reference)
import jax
import jax.numpy as jnp
from jax.experimental import pallas as pl


def kernel(x, edge_index, edge_type, W_embed, b_embed, V1, comb1, Wself1, bias1, V2, comb2, Wself2, bias2):
    raise NotImplementedError("write your pallas kernel here")


# speedup vs baseline: 4.3495x; 4.3495x over previous
import functools

import jax
import jax.numpy as jnp
from jax import lax
from jax.experimental import pallas as pl
from jax.experimental.pallas import tpu as pltpu
from jax.experimental.pallas import tpu_sc as plsc

N = 10000
E = 160000
H = 256
OUT = 64
R = 3
B = 2

NC = 2
NS = 16
K = 80
EROWS = E // K
TROWS = EROWS // NS
CW = 32
CH = H // CW
CNT_PAD = 32768
CNT_TW = CNT_PAD // NS
WR_FULL = 632
WR_LAST = N - (NS - 1) * WR_FULL

_f32 = jnp.float32
_i32 = jnp.int32


def _sc_body(compute_cnt, table, src_hbm, dst_hbm, typ_hbm, s_out, *rest):
    if compute_cnt:
        cnt_out = rest[0]
        rest = rest[1:]
    (src_v, dst_v, typ_v, gidx_v, bkt_v, rows_v, zeros_v, hist_v,
     acc_sh, sem) = rest

    c = lax.axis_index("c")
    s = lax.axis_index("s")
    rbase = s * TROWS

    pltpu.sync_copy(src_hbm.at[pl.ds(rbase, TROWS), :], src_v)
    pltpu.sync_copy(dst_hbm.at[pl.ds(rbase, TROWS), :], dst_v)
    pltpu.sync_copy(typ_hbm.at[pl.ds(rbase, TROWS), :], typ_v)

    z16 = jnp.zeros((16,), _f32)

    @pl.loop(0, 125)
    def _(i):
        for g in range(CW // 16):
            zeros_v[i, pl.ds(g * 16, 16)] = z16

    @pl.loop(0, TROWS)
    def _(i):
        for g in range(K // 16):
            d16 = dst_v[i, pl.ds(g * 16, 16)]
            t16 = typ_v[i, pl.ds(g * 16, 16)]
            bkt_v[i, pl.ds(g * 16, 16)] = t16 * N + d16

    if compute_cnt:
        @pl.when(c == 0)
        def _():
            @pl.loop(0, CNT_TW // 16)
            def _(i):
                hist_v[pl.ds(i * 16, 16)] = z16

            lo = s * CNT_TW
            ones16 = jnp.full((16,), 1.0, _f32)

            @pl.loop(0, NS)
            def _(t):
                pltpu.sync_copy(dst_hbm.at[pl.ds(t * TROWS, TROWS), :],
                                dst_v)
                pltpu.sync_copy(typ_hbm.at[pl.ds(t * TROWS, TROWS), :],
                                typ_v)

                @pl.loop(0, TROWS)
                def _(i):
                    for g in range(K // 16):
                        d16 = dst_v[i, pl.ds(g * 16, 16)]
                        t16 = typ_v[i, pl.ds(g * 16, 16)]
                        b16 = t16 * N + d16 - lo
                        m = (b16 >= 0) & (b16 < CNT_TW)
                        idx = jnp.where(m, b16, 0)
                        plsc.addupdate_scatter(hist_v, [idx], ones16,
                                               mask=m)

            pltpu.sync_copy(hist_v, cnt_out.at[pl.ds(lo, CNT_TW)])

    for kc in range(CH // NC):
        ch = c * (CH // NC) + kc

        plsc.subcore_barrier()

        @pl.loop(0, 15)
        def _(i):
            pltpu.sync_copy(zeros_v,
                            acc_sh.at[pl.ds(s * 1875 + i * 125, 125)])

        plsc.subcore_barrier()

        @pl.loop(0, TROWS)
        def _(i):
            for g in range(K // 16):
                s16 = src_v[i, pl.ds(g * 16, 16)]
                gidx_v[i, pl.ds(g * 16, 16)] = s16 * CH + ch

        @pl.loop(0, TROWS)
        def _(jb):
            pltpu.async_copy(table.at[gidx_v.at[jb]], rows_v, sem).wait()
            pltpu.sync_copy(rows_v, acc_sh.at[bkt_v.at[jb]], add=True)

        plsc.subcore_barrier()

        for r in range(R):
            @pl.when(s < NS - 1)
            def _():
                pltpu.sync_copy(
                    acc_sh.at[pl.ds(r * N + s * WR_FULL, WR_FULL)],
                    s_out.at[r, ch, pl.ds(s * WR_FULL, WR_FULL), :])

            @pl.when(s == NS - 1)
            def _():
                pltpu.sync_copy(
                    acc_sh.at[pl.ds(r * N + (NS - 1) * WR_FULL, WR_LAST)],
                    s_out.at[r, ch, pl.ds((NS - 1) * WR_FULL, WR_LAST), :])


def _sc_segsum(table, src2, dst2, typ2, compute_cnt):
    mesh = plsc.VectorSubcoreMesh(core_axis_name="c", subcore_axis_name="s")
    outs = [jax.ShapeDtypeStruct((R, CH, N, CW), _f32)]
    if compute_cnt:
        outs.append(jax.ShapeDtypeStruct((CNT_PAD,), _f32))
    scratch = [
        pltpu.VMEM((TROWS, K), _i32),
        pltpu.VMEM((TROWS, K), _i32),
        pltpu.VMEM((TROWS, K), _i32),
        pltpu.VMEM((TROWS, K), _i32),
        pltpu.VMEM((TROWS, K), _i32),
        pltpu.VMEM((K, CW), _f32),
        pltpu.VMEM((125, CW), _f32),
        pltpu.VMEM((CNT_TW,), _f32),
        pltpu.VMEM_SHARED((R * N, CW), _f32),
        pltpu.SemaphoreType.DMA,
    ]
    fn = pl.kernel(
        functools.partial(_sc_body, compute_cnt),
        out_type=tuple(outs),
        mesh=mesh,
        scratch_types=scratch,
        compiler_params=pltpu.CompilerParams(needs_layout_passes=False,
                                             use_tc_tiling_on_sc=False),
    )
    res = fn(table, src2, dst2, typ2)
    return res if compute_cnt else res[0]


_BN = 1000


def _p1_body(x_ref, w_ref, b_ref, o_ref):
    o_ref[...] = jnp.dot(x_ref[...], w_ref[...],
                         preferred_element_type=_f32) + b_ref[...]


def _embed(x, w, b):
    d_in, h = w.shape
    return pl.pallas_call(
        _p1_body,
        grid=(N // _BN,),
        in_specs=[
            pl.BlockSpec((_BN, d_in), lambda i: (i, 0)),
            pl.BlockSpec((d_in, h), lambda i: (0, 0)),
            pl.BlockSpec((1, h), lambda i: (0, 0)),
        ],
        out_specs=pl.BlockSpec((_BN, h), lambda i: (i, 0)),
        out_shape=jax.ShapeDtypeStruct((N, h), _f32),
    )(x, w, b)


def _p3_body(relu, s_ref, h_ref, cnt_ref, comb_ref, v_ref, ws_ref, b_ref,
             o_ref):
    icnt = 1.0 / jnp.maximum(cnt_ref[...], 1.0)
    comb = comb_ref[...]
    hh = h_ref[...]
    acc = jnp.dot(hh, ws_ref[...], preferred_element_type=_f32)
    for b in range(B):
        for cch in range(CH):
            cb = jnp.zeros((hh.shape[0], CW), _f32)
            for r in range(R):
                cb = cb + s_ref[r, cch] * (icnt[:, r:r + 1] * comb[r, b])
            acc = acc + jnp.dot(cb, v_ref[b, cch],
                                preferred_element_type=_f32)
    acc = acc + b_ref[...]
    if relu:
        acc = jnp.maximum(acc, 0.0)
    o_ref[...] = acc


def _combine(S4, h, cntN3, comb, V, Wself, bias, relu):
    out_dim = V.shape[2]
    v4 = V.reshape(B, CH, CW, out_dim)
    return pl.pallas_call(
        functools.partial(_p3_body, relu),
        grid=(N // _BN,),
        in_specs=[
            pl.BlockSpec((R, CH, _BN, CW), lambda i: (0, 0, i, 0)),
            pl.BlockSpec((_BN, H), lambda i: (i, 0)),
            pl.BlockSpec((_BN, R), lambda i: (i, 0)),
            pl.BlockSpec((R, B), lambda i: (0, 0)),
            pl.BlockSpec((B, CH, CW, out_dim), lambda i: (0, 0, 0, 0)),
            pl.BlockSpec((H, out_dim), lambda i: (0, 0)),
            pl.BlockSpec((1, out_dim), lambda i: (0, 0)),
        ],
        out_specs=pl.BlockSpec((_BN, out_dim), lambda i: (i, 0)),
        out_shape=jax.ShapeDtypeStruct((N, out_dim), _f32),
    )(S4, h, cntN3, comb, v4, Wself, bias)


def kernel(x, edge_index, edge_type, W_embed, b_embed, V1, comb1, Wself1,
           bias1, V2, comb2, Wself2, bias2):
    src2 = edge_index[0].astype(_i32).reshape(EROWS, K)
    dst2 = edge_index[1].astype(_i32).reshape(EROWS, K)
    typ2 = edge_type.astype(_i32).reshape(EROWS, K)

    h0 = _embed(x, W_embed, b_embed.reshape(1, -1))

    S1, cnt = _sc_segsum(h0.reshape(CH * N, CW), src2, dst2, typ2, True)
    cntN3 = cnt[:R * N].reshape(R, N).T

    h1 = _combine(S1, h0, cntN3, comb1, V1, Wself1,
                  bias1.reshape(1, -1), True)

    S2 = _sc_segsum(h1.reshape(CH * N, CW), src2, dst2, typ2, False)

    out = _combine(S2, h1, cntN3, comb2, V2, Wself2,
                   bias2.reshape(1, -1), False)
    return out


# speedup vs baseline: 4.9690x; 1.1424x over previous
import functools

import jax
import jax.numpy as jnp
from jax import lax
from jax.experimental import pallas as pl
from jax.experimental.pallas import tpu as pltpu
from jax.experimental.pallas import tpu_sc as plsc

N = 10000
E = 160000
H = 256
OUT = 64
R = 3
B = 2

NC = 2
NS = 16
K = 128
EP = 163840
EROWS = EP // K
TROWS = EROWS // NS
G = 8
ACC_ROWS = 30080
CW = 32
CH = H // CW
CNT_PAD = 32768
CNT_TW = CNT_PAD // NS
WR_FULL = 632
WR_LAST = N - (NS - 1) * WR_FULL

_f32 = jnp.float32
_i32 = jnp.int32


def _sc_body(compute_cnt, table, src_hbm, dst_hbm, typ_hbm, s_out, *rest):
    if compute_cnt:
        cnt_out, bkt_hbm = rest[0], rest[1]
        rest = rest[2:]
    (src_v, bkt_v, gidx_g, rows_v, zeros_v, hist_v, dst8_v, typ8_v,
     acc_sh, semg, sems) = rest

    c = lax.axis_index("c")
    s = lax.axis_index("s")
    rbase = s * TROWS

    pltpu.sync_copy(src_hbm.at[pl.ds(rbase, TROWS), :], src_v)

    z16 = jnp.zeros((16,), _f32)

    @pl.loop(0, 94)
    def _(i):
        for g in range(CW // 16):
            zeros_v[i, pl.ds(g * 16, 16)] = z16

    @pl.loop(0, TROWS // 8)
    def _(o):
        pltpu.sync_copy(dst_hbm.at[pl.ds(rbase + o * 8, 8), :], dst8_v)
        pltpu.sync_copy(typ_hbm.at[pl.ds(rbase + o * 8, 8), :], typ8_v)
        for i in range(8):
            for g in range(K // 16):
                d16 = dst8_v[i, pl.ds(g * 16, 16)]
                t16 = typ8_v[i, pl.ds(g * 16, 16)]
                bkt_v[o * 8 + i, pl.ds(g * 16, 16)] = t16 * N + d16

    if compute_cnt:
        @pl.when(c == 0)
        def _():
            pltpu.sync_copy(bkt_v, bkt_hbm.at[pl.ds(rbase, TROWS), :])
            plsc.subcore_barrier()

            @pl.loop(0, CNT_TW // 16)
            def _(i):
                hist_v[pl.ds(i * 16, 16)] = z16

            lo = s * CNT_TW
            ones16 = jnp.full((16,), 1.0, _f32)

            @pl.loop(0, NS)
            def _(t):
                pltpu.sync_copy(bkt_hbm.at[pl.ds(t * TROWS, TROWS), :],
                                bkt_v)

                @pl.loop(0, TROWS)
                def _(i):
                    for g in range(K // 16):
                        b16 = bkt_v[i, pl.ds(g * 16, 16)] - lo
                        m = (b16 >= 0) & (b16 < CNT_TW)
                        idx = jnp.where(m, b16, 0)
                        plsc.addupdate_scatter(hist_v, [idx], ones16,
                                               mask=m)

            pltpu.sync_copy(hist_v, cnt_out.at[pl.ds(lo, CNT_TW)])
            pltpu.sync_copy(bkt_hbm.at[pl.ds(rbase, TROWS), :], bkt_v)

    for kc in range(CH // NC):
        ch = c * (CH // NC) + kc

        plsc.subcore_barrier()

        @pl.loop(0, 20)
        def _(i):
            pltpu.sync_copy(zeros_v,
                            acc_sh.at[pl.ds(s * 1880 + i * 94, 94)])

        plsc.subcore_barrier()

        @pl.loop(0, TROWS // G)
        def _(o):
            jb0 = o * G
            for g in range(G):
                for q in range(K // 16):
                    s16 = src_v[jb0 + g, pl.ds(q * 16, 16)]
                    gidx_g[pl.ds(g * K + q * 16, 16)] = s16 * CH + ch
            gds = [pltpu.async_copy(table.at[gidx_g.at[pl.ds(g * K, K)]],
                                    rows_v.at[pl.ds(g * K, K)], semg)
                   for g in range(G)]
            for d in gds:
                d.wait()
            sds = [pltpu.async_copy(rows_v.at[pl.ds(g * K, K)],
                                    acc_sh.at[bkt_v.at[jb0 + g]], sems,
                                    add=True)
                   for g in range(G)]
            for d in sds:
                d.wait()

        plsc.subcore_barrier()

        for r in range(R):
            @pl.when(s < NS - 1)
            def _():
                pltpu.sync_copy(
                    acc_sh.at[pl.ds(r * N + s * WR_FULL, WR_FULL)],
                    s_out.at[r, ch, pl.ds(s * WR_FULL, WR_FULL), :])

            @pl.when(s == NS - 1)
            def _():
                pltpu.sync_copy(
                    acc_sh.at[pl.ds(r * N + (NS - 1) * WR_FULL, WR_LAST)],
                    s_out.at[r, ch, pl.ds((NS - 1) * WR_FULL, WR_LAST), :])


def _sc_segsum(table, src2, dst2, typ2, compute_cnt):
    mesh = plsc.VectorSubcoreMesh(core_axis_name="c", subcore_axis_name="s")
    outs = [jax.ShapeDtypeStruct((R, CH, N, CW), _f32)]
    if compute_cnt:
        outs.append(jax.ShapeDtypeStruct((CNT_PAD,), _f32))
        outs.append(jax.ShapeDtypeStruct((EROWS, K), _i32))
    scratch = [
        pltpu.VMEM((TROWS, K), _i32),
        pltpu.VMEM((TROWS, K), _i32),
        pltpu.VMEM((G * K,), _i32),
        pltpu.VMEM((G * K, CW), _f32),
        pltpu.VMEM((94, CW), _f32),
        pltpu.VMEM((CNT_TW,), _f32),
        pltpu.VMEM((8, K), _i32),
        pltpu.VMEM((8, K), _i32),
        pltpu.VMEM_SHARED((ACC_ROWS, CW), _f32),
        pltpu.SemaphoreType.DMA,
        pltpu.SemaphoreType.DMA,
    ]
    fn = pl.kernel(
        functools.partial(_sc_body, compute_cnt),
        out_type=tuple(outs),
        mesh=mesh,
        scratch_types=scratch,
        compiler_params=pltpu.CompilerParams(needs_layout_passes=False,
                                             use_tc_tiling_on_sc=False),
    )
    res = fn(table, src2, dst2, typ2)
    return (res[0], res[1]) if compute_cnt else res[0]


_BN = 1000


def _p1_body(x_ref, w_ref, b_ref, o_ref):
    o_ref[...] = jnp.dot(x_ref[...], w_ref[...],
                         preferred_element_type=_f32) + b_ref[...]


def _embed(x, w, b):
    d_in, h = w.shape
    return pl.pallas_call(
        _p1_body,
        grid=(N // _BN,),
        in_specs=[
            pl.BlockSpec((_BN, d_in), lambda i: (i, 0)),
            pl.BlockSpec((d_in, h), lambda i: (0, 0)),
            pl.BlockSpec((1, h), lambda i: (0, 0)),
        ],
        out_specs=pl.BlockSpec((_BN, h), lambda i: (i, 0)),
        out_shape=jax.ShapeDtypeStruct((N, h), _f32),
    )(x, w, b)


def _p3_body(relu, s_ref, h_ref, cnt_ref, comb_ref, v_ref, ws_ref, b_ref,
             o_ref):
    icnt = 1.0 / jnp.maximum(cnt_ref[...], 1.0)
    comb = comb_ref[...]
    hh = h_ref[...]
    acc = jnp.dot(hh, ws_ref[...], preferred_element_type=_f32)
    for b in range(B):
        for cch in range(CH):
            cb = jnp.zeros((hh.shape[0], CW), _f32)
            for r in range(R):
                cb = cb + s_ref[r, cch] * (icnt[:, r:r + 1] * comb[r, b])
            acc = acc + jnp.dot(cb, v_ref[b, cch],
                                preferred_element_type=_f32)
    acc = acc + b_ref[...]
    if relu:
        acc = jnp.maximum(acc, 0.0)
    o_ref[...] = acc


def _combine(S4, h, cntN3, comb, V, Wself, bias, relu):
    out_dim = V.shape[2]
    v4 = V.reshape(B, CH, CW, out_dim)
    return pl.pallas_call(
        functools.partial(_p3_body, relu),
        grid=(N // _BN,),
        in_specs=[
            pl.BlockSpec((R, CH, _BN, CW), lambda i: (0, 0, i, 0)),
            pl.BlockSpec((_BN, H), lambda i: (i, 0)),
            pl.BlockSpec((_BN, R), lambda i: (i, 0)),
            pl.BlockSpec((R, B), lambda i: (0, 0)),
            pl.BlockSpec((B, CH, CW, out_dim), lambda i: (0, 0, 0, 0)),
            pl.BlockSpec((H, out_dim), lambda i: (0, 0)),
            pl.BlockSpec((1, out_dim), lambda i: (0, 0)),
        ],
        out_specs=pl.BlockSpec((_BN, out_dim), lambda i: (i, 0)),
        out_shape=jax.ShapeDtypeStruct((N, out_dim), _f32),
    )(S4, h, cntN3, comb, v4, Wself, bias)


def kernel(x, edge_index, edge_type, W_embed, b_embed, V1, comb1, Wself1,
           bias1, V2, comb2, Wself2, bias2):
    npad = EP - E
    src2 = jnp.concatenate(
        [edge_index[0].astype(_i32), jnp.zeros((npad,), _i32)]
    ).reshape(EROWS, K)
    dst2 = jnp.concatenate(
        [edge_index[1].astype(_i32), jnp.zeros((npad,), _i32)]
    ).reshape(EROWS, K)
    typ2 = jnp.concatenate(
        [edge_type.astype(_i32), jnp.full((npad,), R, _i32)]
    ).reshape(EROWS, K)

    h0 = _embed(x, W_embed, b_embed.reshape(1, -1))

    S1, cnt = _sc_segsum(h0.reshape(CH * N, CW), src2, dst2, typ2, True)
    cntN3 = cnt[:R * N].reshape(R, N).T

    h1 = _combine(S1, h0, cntN3, comb1, V1, Wself1,
                  bias1.reshape(1, -1), True)

    S2 = _sc_segsum(h1.reshape(CH * N, CW), src2, dst2, typ2, False)

    out = _combine(S2, h1, cntN3, comb2, V2, Wself2,
                   bias2.reshape(1, -1), False)
    return out


# speedup vs baseline: 5.7697x; 1.1611x over previous
import functools

import jax
import jax.numpy as jnp
from jax import lax
from jax.experimental import pallas as pl
from jax.experimental.pallas import tpu as pltpu
from jax.experimental.pallas import tpu_sc as plsc

N = 10000
E = 160000
H = 256
OUT = 64
R = 3
B = 2

NC = 2
NS = 16
K = 128
EP = 163840
EROWS = EP // K
TROWS = EROWS // NS
G = 4
ACC_ROWS = 30080
CW = 32
CH = H // CW
CNT_PAD = 32768
CNT_TW = CNT_PAD // NS
WR_FULL = 632
WR_LAST = N - (NS - 1) * WR_FULL

_f32 = jnp.float32
_i32 = jnp.int32


def _sc_body(compute_cnt, table, src_hbm, dst_hbm, typ_hbm, s_out, *rest):
    if compute_cnt:
        cnt_out, bkt_hbm = rest[0], rest[1]
        rest = rest[2:]
    (src_v, bkt_v, gidx2_v, rows2_v, zeros_v, hist_v, dst8_v, typ8_v,
     acc_sh, semg0, semg1, sems0, sems1) = rest
    semg = [semg0, semg1]
    sems = [sems0, sems1]

    c = lax.axis_index("c")
    s = lax.axis_index("s")
    rbase = s * TROWS

    pltpu.sync_copy(src_hbm.at[pl.ds(rbase, TROWS), :], src_v)

    z16 = jnp.zeros((16,), _f32)

    @pl.loop(0, 235)
    def _(i):
        for g in range(CW // 16):
            zeros_v[i, pl.ds(g * 16, 16)] = z16

    @pl.loop(0, TROWS // 8)
    def _(o):
        pltpu.sync_copy(dst_hbm.at[pl.ds(rbase + o * 8, 8), :], dst8_v)
        pltpu.sync_copy(typ_hbm.at[pl.ds(rbase + o * 8, 8), :], typ8_v)
        for i in range(8):
            for g in range(K // 16):
                d16 = dst8_v[i, pl.ds(g * 16, 16)]
                t16 = typ8_v[i, pl.ds(g * 16, 16)]
                bkt_v[o * 8 + i, pl.ds(g * 16, 16)] = t16 * N + d16

    if compute_cnt:
        pltpu.sync_copy(bkt_v, bkt_hbm.at[c, pl.ds(rbase, TROWS), :])
        plsc.subcore_barrier()

        @pl.loop(0, CNT_TW // 16)
        def _(i):
            hist_v[pl.ds(i * 16, 16)] = z16

        lo = s * CNT_TW
        ones16 = jnp.full((16,), 1.0, _f32)
        half = EROWS // NC

        @pl.loop(0, half // TROWS)
        def _(t):
            pltpu.sync_copy(
                bkt_hbm.at[c, pl.ds(c * half + t * TROWS, TROWS), :],
                bkt_v)

            @pl.loop(0, TROWS)
            def _(i):
                for g in range(K // 16):
                    b16 = bkt_v[i, pl.ds(g * 16, 16)] - lo
                    m = (b16 >= 0) & (b16 < CNT_TW)
                    idx = jnp.where(m, b16, 0)
                    plsc.addupdate_scatter(hist_v, [idx], ones16, mask=m)

        pltpu.sync_copy(hist_v, cnt_out.at[c, pl.ds(lo, CNT_TW)])
        pltpu.sync_copy(bkt_hbm.at[c, pl.ds(rbase, TROWS), :], bkt_v)

    NG = TROWS // G

    def compute_gidx(a, g, ch):
        for q in range(G):
            for w in range(K // 16):
                s16 = src_v[g * G + q, pl.ds(w * 16, 16)]
                gidx2_v[a, pl.ds(q * K + w * 16, 16)] = s16 * CH + ch

    def issue_gathers(a, g):
        for q in range(G):
            pltpu.async_copy(
                table.at[gidx2_v.at[a, pl.ds(q * K, K)]],
                rows2_v.at[pl.ds((a * G + q) * K, K)], semg[a])

    def drain_gathers(a):
        for q in range(G):
            pltpu.make_async_copy(
                table.at[gidx2_v.at[a, pl.ds(q * K, K)]],
                rows2_v.at[pl.ds((a * G + q) * K, K)], semg[a]).wait()

    for kc in range(CH // NC):
        ch = c * (CH // NC) + kc

        plsc.subcore_barrier()

        compute_gidx(0, 0, ch)
        issue_gathers(0, 0)

        @pl.loop(0, 8)
        def _(i):
            pltpu.sync_copy(zeros_v,
                            acc_sh.at[pl.ds(s * 1880 + i * 235, 235)])

        plsc.subcore_barrier()

        @pl.loop(0, NG // 2)
        def _(pp):
            for a in range(2):
                g = pp * 2 + a
                nxt = 1 - a

                @pl.when(g + 1 < NG)
                def _():
                    compute_gidx(nxt, g + 1, ch)
                    issue_gathers(nxt, g + 1)

                drain_gathers(a)
                sds = [pltpu.async_copy(
                           rows2_v.at[pl.ds((a * G + q) * K, K)],
                           acc_sh.at[bkt_v.at[g * G + q]], sems[a],
                           add=True)
                       for q in range(G)]
                for d in sds:
                    d.wait()

        plsc.subcore_barrier()

        for r in range(R):
            @pl.when(s < NS - 1)
            def _():
                pltpu.sync_copy(
                    acc_sh.at[pl.ds(r * N + s * WR_FULL, WR_FULL)],
                    s_out.at[r, ch, pl.ds(s * WR_FULL, WR_FULL), :])

            @pl.when(s == NS - 1)
            def _():
                pltpu.sync_copy(
                    acc_sh.at[pl.ds(r * N + (NS - 1) * WR_FULL, WR_LAST)],
                    s_out.at[r, ch, pl.ds((NS - 1) * WR_FULL, WR_LAST), :])


def _sc_segsum(table, src2, dst2, typ2, compute_cnt):
    mesh = plsc.VectorSubcoreMesh(core_axis_name="c", subcore_axis_name="s")
    outs = [jax.ShapeDtypeStruct((R, CH, N, CW), _f32)]
    if compute_cnt:
        outs.append(jax.ShapeDtypeStruct((NC, CNT_PAD), _f32))
        outs.append(jax.ShapeDtypeStruct((NC, EROWS, K), _i32))
    scratch = [
        pltpu.VMEM((TROWS, K), _i32),
        pltpu.VMEM((TROWS, K), _i32),
        pltpu.VMEM((2, G * K), _i32),
        pltpu.VMEM((2 * G * K, CW), _f32),
        pltpu.VMEM((235, CW), _f32),
        pltpu.VMEM((CNT_TW,), _f32),
        pltpu.VMEM((8, K), _i32),
        pltpu.VMEM((8, K), _i32),
        pltpu.VMEM_SHARED((ACC_ROWS, CW), _f32),
        pltpu.SemaphoreType.DMA,
        pltpu.SemaphoreType.DMA,
        pltpu.SemaphoreType.DMA,
        pltpu.SemaphoreType.DMA,
    ]
    fn = pl.kernel(
        functools.partial(_sc_body, compute_cnt),
        out_type=tuple(outs),
        mesh=mesh,
        scratch_types=scratch,
        compiler_params=pltpu.CompilerParams(needs_layout_passes=False,
                                             use_tc_tiling_on_sc=False),
    )
    res = fn(table, src2, dst2, typ2)
    return (res[0], res[1]) if compute_cnt else res[0]


_BN = 1000


def _p1_body(x_ref, w_ref, b_ref, o_ref):
    o_ref[...] = jnp.dot(x_ref[...], w_ref[...],
                         preferred_element_type=_f32) + b_ref[...]


def _embed(x, w, b):
    d_in, h = w.shape
    return pl.pallas_call(
        _p1_body,
        grid=(N // _BN,),
        in_specs=[
            pl.BlockSpec((_BN, d_in), lambda i: (i, 0)),
            pl.BlockSpec((d_in, h), lambda i: (0, 0)),
            pl.BlockSpec((1, h), lambda i: (0, 0)),
        ],
        out_specs=pl.BlockSpec((_BN, h), lambda i: (i, 0)),
        out_shape=jax.ShapeDtypeStruct((N, h), _f32),
    )(x, w, b)


def _p3_body(relu, s_ref, h_ref, cnt_ref, comb_ref, v_ref, ws_ref, b_ref,
             o_ref):
    icnt = 1.0 / jnp.maximum(cnt_ref[0] + cnt_ref[1], 1.0)
    comb = comb_ref[...]
    hh = h_ref[...]
    acc = jnp.dot(hh, ws_ref[...], preferred_element_type=_f32)
    for b in range(B):
        for cch in range(CH):
            cb = jnp.zeros((hh.shape[0], CW), _f32)
            for r in range(R):
                cb = cb + s_ref[r, cch] * (icnt[:, r:r + 1] * comb[r, b])
            acc = acc + jnp.dot(cb, v_ref[b, cch],
                                preferred_element_type=_f32)
    acc = acc + b_ref[...]
    if relu:
        acc = jnp.maximum(acc, 0.0)
    o_ref[...] = acc


def _combine(S4, h, cntN3, comb, V, Wself, bias, relu):
    out_dim = V.shape[2]
    v4 = V.reshape(B, CH, CW, out_dim)
    return pl.pallas_call(
        functools.partial(_p3_body, relu),
        grid=(N // _BN,),
        in_specs=[
            pl.BlockSpec((R, CH, _BN, CW), lambda i: (0, 0, i, 0)),
            pl.BlockSpec((_BN, H), lambda i: (i, 0)),
            pl.BlockSpec((NC, _BN, R), lambda i: (0, i, 0)),
            pl.BlockSpec((R, B), lambda i: (0, 0)),
            pl.BlockSpec((B, CH, CW, out_dim), lambda i: (0, 0, 0, 0)),
            pl.BlockSpec((H, out_dim), lambda i: (0, 0)),
            pl.BlockSpec((1, out_dim), lambda i: (0, 0)),
        ],
        out_specs=pl.BlockSpec((_BN, out_dim), lambda i: (i, 0)),
        out_shape=jax.ShapeDtypeStruct((N, out_dim), _f32),
    )(S4, h, cntN3, comb, v4, Wself, bias)


def kernel(x, edge_index, edge_type, W_embed, b_embed, V1, comb1, Wself1,
           bias1, V2, comb2, Wself2, bias2):
    npad = EP - E
    src2 = jnp.concatenate(
        [edge_index[0].astype(_i32), jnp.zeros((npad,), _i32)]
    ).reshape(EROWS, K)
    dst2 = jnp.concatenate(
        [edge_index[1].astype(_i32), jnp.zeros((npad,), _i32)]
    ).reshape(EROWS, K)
    typ2 = jnp.concatenate(
        [edge_type.astype(_i32), jnp.full((npad,), R, _i32)]
    ).reshape(EROWS, K)

    h0 = _embed(x, W_embed, b_embed.reshape(1, -1))

    S1, cnt = _sc_segsum(h0.reshape(CH * N, CW), src2, dst2, typ2, True)
    cntN3 = cnt[:, :R * N].reshape(NC, R, N).transpose(0, 2, 1)

    h1 = _combine(S1, h0, cntN3, comb1, V1, Wself1,
                  bias1.reshape(1, -1), True)

    S2 = _sc_segsum(h1.reshape(CH * N, CW), src2, dst2, typ2, False)

    out = _combine(S2, h1, cntN3, comb2, V2, Wself2,
                   bias2.reshape(1, -1), False)
    return out
